# baseline (speedup 1.0000x reference)
import functools

import jax
import jax.numpy as jnp
from jax import lax
from jax.experimental import pallas as pl
from jax.experimental.pallas import tpu as pltpu
from jax.experimental.pallas import tpu_sc as plsc

N_ENT = 100000
N_REL = 1000
D = 64
B = 16384
MULT = 20.0

NC = 2
NS = 16
L = 16
NW = NC * NS
BPW = B // NW
CH = 128
NCHUNK = BPW // CH
NG = CH // L

_mesh = plsc.VectorSubcoreMesh(core_axis_name="c", subcore_axis_name="s")


@functools.partial(
    pl.kernel,
    out_type=jax.ShapeDtypeStruct((B,), jnp.float32),
    mesh=_mesh,
    compiler_params=pltpu.CompilerParams(
        needs_layout_passes=False, use_tc_tiling_on_sc=False),
    scratch_types=[
        pltpu.VMEM((CH,), jnp.int32),
        pltpu.VMEM((CH,), jnp.int32),
        pltpu.VMEM((CH,), jnp.int32),
        pltpu.VMEM((CH, D), jnp.float32),
        pltpu.VMEM((CH, D), jnp.float32),
        pltpu.VMEM((CH, D), jnp.float32),
        pltpu.VMEM((CH, D), jnp.float32),
        pltpu.VMEM((CH, D), jnp.float32),
        pltpu.VMEM((CH, D), jnp.float32),
        pltpu.VMEM((CH, D), jnp.float32),
        pltpu.VMEM((CH,), jnp.float32),
        pltpu.SemaphoreType.DMA,
    ],
)
def _sc_score(s_hbm, r_hbm, o_hbm, e_hbm, rr_hbm, et_hbm, rht_hbm, rtt_hbm,
              out_hbm,
              sidx, ridx, oidx, se, re, oe, st, rht, rtt, ot, outv, sem):
    wid = lax.axis_index("s") * NC + lax.axis_index("c")

    def chunk_body(c, carry):
        base = pl.multiple_of(wid * BPW + c * CH, CH)
        pltpu.sync_copy(s_hbm.at[pl.ds(base, CH)], sidx)
        pltpu.sync_copy(r_hbm.at[pl.ds(base, CH)], ridx)
        pltpu.sync_copy(o_hbm.at[pl.ds(base, CH)], oidx)
        cps = [
            pltpu.async_copy(e_hbm.at[sidx], se, sem),
            pltpu.async_copy(rr_hbm.at[ridx], re, sem),
            pltpu.async_copy(e_hbm.at[oidx], oe, sem),
            pltpu.async_copy(et_hbm.at[sidx], st, sem),
            pltpu.async_copy(rht_hbm.at[ridx], rht, sem),
            pltpu.async_copy(rtt_hbm.at[ridx], rtt, sem),
            pltpu.async_copy(et_hbm.at[oidx], ot, sem),
        ]
        for cp in cps:
            cp.wait()

        for g in range(NG):
            tvec = lax.iota(jnp.int32, 16) + g * L

            def dim_body(d, accs):
                b_acc, h_acc, t_acc = accs
                dv = jnp.full((L,), d, jnp.int32)
                sev = plsc.load_gather(se, [tvec, dv])
                rev = plsc.load_gather(re, [tvec, dv])
                oev = plsc.load_gather(oe, [tvec, dv])
                stv = plsc.load_gather(st, [tvec, dv])
                rhtv = plsc.load_gather(rht, [tvec, dv])
                rttv = plsc.load_gather(rtt, [tvec, dv])
                otv = plsc.load_gather(ot, [tvec, dv])
                return (b_acc + sev * rev * oev,
                        h_acc + stv * rhtv,
                        t_acc + otv * rttv)

            z = jnp.zeros((L,), jnp.float32)
            b_acc, h_acc, t_acc = lax.fori_loop(0, D, dim_body, (z, z, z))
            res = (MULT
                   / (1.0 + jnp.exp(-b_acc))
                   / (1.0 + jnp.exp(-h_acc))
                   / (1.0 + jnp.exp(-t_acc)))
            outv[pl.ds(g * L, L)] = res

        pltpu.sync_copy(outv, out_hbm.at[pl.ds(base, CH)])
        return carry

    lax.fori_loop(0, NCHUNK, chunk_body, 0)


def kernel(s, r, o, E, R, E_t, R_ht, R_tt):
    return _sc_score(s, r, o, E, R, E_t, R_ht, R_tt)


# speedup vs baseline: 1.7251x; 1.7251x over previous
import functools

import jax
import jax.numpy as jnp
from jax import lax
from jax.experimental import pallas as pl
from jax.experimental.pallas import tpu as pltpu
from jax.experimental.pallas import tpu_sc as plsc

N_ENT = 100000
N_REL = 1000
D = 64
B = 16384
MULT = 20.0

NC = 2
NS = 16
L = 16
NW = NC * NS
BPW = B // NW
CH = 128
NCHUNK = BPW // CH
NG = CH // L

_mesh = plsc.VectorSubcoreMesh(core_axis_name="c", subcore_axis_name="s")


@functools.partial(
    pl.kernel,
    out_type=jax.ShapeDtypeStruct((B,), jnp.float32),
    mesh=_mesh,
    compiler_params=pltpu.CompilerParams(
        needs_layout_passes=False, use_tc_tiling_on_sc=False),
    scratch_types=[
        pltpu.VMEM((CH,), jnp.int32),
        pltpu.VMEM((CH,), jnp.int32),
        pltpu.VMEM((CH,), jnp.int32),
        pltpu.VMEM((CH, D), jnp.float32),
        pltpu.VMEM((CH, D), jnp.float32),
        pltpu.VMEM((CH, D), jnp.float32),
        pltpu.VMEM((CH, D), jnp.float32),
        pltpu.VMEM((CH, D), jnp.float32),
        pltpu.VMEM((CH, D), jnp.float32),
        pltpu.VMEM((CH, D), jnp.float32),
        pltpu.VMEM((CH,), jnp.float32),
        pltpu.SemaphoreType.DMA,
    ],
)
def _sc_score(s_hbm, r_hbm, o_hbm, e_hbm, rr_hbm, et_hbm, rht_hbm, rtt_hbm,
              out_hbm,
              sidx, ridx, oidx, se, re, oe, st, rht, rtt, ot, outv, sem):
    wid = lax.axis_index("s") * NC + lax.axis_index("c")

    def chunk_body(c, carry):
        base = pl.multiple_of(wid * BPW + c * CH, CH)
        pltpu.sync_copy(s_hbm.at[pl.ds(base, CH)], sidx)
        pltpu.sync_copy(r_hbm.at[pl.ds(base, CH)], ridx)
        pltpu.sync_copy(o_hbm.at[pl.ds(base, CH)], oidx)
        cps = [
            pltpu.async_copy(e_hbm.at[sidx], se, sem),
            pltpu.async_copy(rr_hbm.at[ridx], re, sem),
            pltpu.async_copy(e_hbm.at[oidx], oe, sem),
            pltpu.async_copy(et_hbm.at[sidx], st, sem),
            pltpu.async_copy(rht_hbm.at[ridx], rht, sem),
            pltpu.async_copy(rtt_hbm.at[ridx], rtt, sem),
            pltpu.async_copy(et_hbm.at[oidx], ot, sem),
        ]
        for cp in cps:
            cp.wait()

        lane = lax.iota(jnp.int32, 16)
        for g in range(NG):
            tvec = lane + g * L

            def dim_body(d, accs):
                b_acc, h_acc, t_acc = accs
                dv = (lane + d) & 63
                sev = plsc.load_gather(se, [tvec, dv])
                rev = plsc.load_gather(re, [tvec, dv])
                oev = plsc.load_gather(oe, [tvec, dv])
                stv = plsc.load_gather(st, [tvec, dv])
                rhtv = plsc.load_gather(rht, [tvec, dv])
                rttv = plsc.load_gather(rtt, [tvec, dv])
                otv = plsc.load_gather(ot, [tvec, dv])
                return (b_acc + sev * rev * oev,
                        h_acc + stv * rhtv,
                        t_acc + otv * rttv)

            z = jnp.zeros((L,), jnp.float32)
            b_acc, h_acc, t_acc = lax.fori_loop(0, D, dim_body, (z, z, z))
            res = (MULT
                   / (1.0 + jnp.exp(-b_acc))
                   / (1.0 + jnp.exp(-h_acc))
                   / (1.0 + jnp.exp(-t_acc)))
            outv[pl.ds(g * L, L)] = res

        pltpu.sync_copy(outv, out_hbm.at[pl.ds(base, CH)])
        return carry

    lax.fori_loop(0, NCHUNK, chunk_body, 0)


def kernel(s, r, o, E, R, E_t, R_ht, R_tt):
    return _sc_score(s, r, o, E, R, E_t, R_ht, R_tt)


# speedup vs baseline: 1.9317x; 1.1197x over previous
import functools

import jax
import jax.numpy as jnp
from jax import lax
from jax.experimental import pallas as pl
from jax.experimental.pallas import tpu as pltpu
from jax.experimental.pallas import tpu_sc as plsc

N_ENT = 100000
N_REL = 1000
D = 64
W = 128
B = 16384
MULT = 20.0

NC = 2
NS = 16
L = 16
NW = NC * NS
BPW = B // NW
CH = 128
NCHUNK = BPW // CH
NG = CH // L

_mesh = plsc.VectorSubcoreMesh(core_axis_name="c", subcore_axis_name="s")


@functools.partial(
    pl.kernel,
    out_type=jax.ShapeDtypeStruct((B,), jnp.float32),
    mesh=_mesh,
    compiler_params=pltpu.CompilerParams(
        needs_layout_passes=False, use_tc_tiling_on_sc=True),
    scratch_types=[
        pltpu.VMEM((CH,), jnp.int32),
        pltpu.VMEM((CH,), jnp.int32),
        pltpu.VMEM((CH,), jnp.int32),
        pltpu.VMEM((CH, W), jnp.float32),
        pltpu.VMEM((CH, W), jnp.float32),
        pltpu.VMEM((CH, W), jnp.float32),
        pltpu.VMEM((CH, W), jnp.float32),
        pltpu.VMEM((CH,), jnp.float32),
        pltpu.SemaphoreType.DMA,
    ],
)
def _sc_score(s_hbm, r_hbm, o_hbm, ee_hbm, rp_hbm, rhtt_hbm,
              out_hbm,
              sidx, ridx, oidx, srow, orow, rrow, rtrow, outv, sem):
    wid = lax.axis_index("s") * NC + lax.axis_index("c")

    def chunk_body(c, carry):
        base = pl.multiple_of(wid * BPW + c * CH, CH)
        pltpu.sync_copy(s_hbm.at[pl.ds(base, CH)], sidx)
        pltpu.sync_copy(r_hbm.at[pl.ds(base, CH)], ridx)
        pltpu.sync_copy(o_hbm.at[pl.ds(base, CH)], oidx)
        cps = [
            pltpu.async_copy(ee_hbm.at[sidx], srow, sem),
            pltpu.async_copy(ee_hbm.at[oidx], orow, sem),
            pltpu.async_copy(rp_hbm.at[ridx], rrow, sem),
            pltpu.async_copy(rhtt_hbm.at[ridx], rtrow, sem),
        ]
        for cp in cps:
            cp.wait()

        lane = lax.iota(jnp.int32, 16)
        for g in range(NG):
            tvec = lane + g * L

            def dim_body(d, accs):
                b_acc, h_acc, t_acc = accs
                dv = (lane + d) & 63
                dv2 = dv + 64
                s_e = plsc.load_gather(srow, [tvec, dv])
                s_t = plsc.load_gather(srow, [tvec, dv2])
                o_e = plsc.load_gather(orow, [tvec, dv])
                o_t = plsc.load_gather(orow, [tvec, dv2])
                r_e = plsc.load_gather(rrow, [tvec, dv])
                r_h = plsc.load_gather(rtrow, [tvec, dv])
                r_t = plsc.load_gather(rtrow, [tvec, dv2])
                return (b_acc + s_e * r_e * o_e,
                        h_acc + s_t * r_h,
                        t_acc + o_t * r_t)

            z = jnp.zeros((L,), jnp.float32)
            b_acc, h_acc, t_acc = lax.fori_loop(0, D, dim_body, (z, z, z))
            res = (MULT
                   / (1.0 + jnp.exp(-b_acc))
                   / (1.0 + jnp.exp(-h_acc))
                   / (1.0 + jnp.exp(-t_acc)))
            outv[pl.ds(g * L, L)] = res

        pltpu.sync_copy(outv, out_hbm.at[pl.ds(base, CH)])
        return carry

    lax.fori_loop(0, NCHUNK, chunk_body, 0)


def kernel(s, r, o, E, R, E_t, R_ht, R_tt):
    ee = jnp.concatenate([E, E_t], axis=1)
    rp = jnp.concatenate([R, jnp.zeros_like(R)], axis=1)
    rhtt = jnp.concatenate([R_ht, R_tt], axis=1)
    return _sc_score(s, r, o, ee, rp, rhtt)
